# baseline (speedup 1.0000x reference)
import jax
import jax.numpy as jnp
from jax.experimental import pallas as pl


def _proj_kernel(a_ref, w_ref, b_ref, o_ref):
    o_ref[...] = jnp.dot(a_ref[...], w_ref[...],
                         preferred_element_type=jnp.float32) + b_ref[...]


def _proj(a, w, b):
    n, k = a.shape
    c = w.shape[1]
    blk = 512
    n_pad = ((n + blk - 1) // blk) * blk
    a_p = jnp.pad(a, ((0, n_pad - n), (0, 0)))
    out = pl.pallas_call(
        _proj_kernel,
        grid=(n_pad // blk,),
        in_specs=[
            pl.BlockSpec((blk, k), lambda i: (i, 0)),
            pl.BlockSpec((k, c), lambda i: (0, 0)),
            pl.BlockSpec((1, c), lambda i: (0, 0)),
        ],
        out_specs=pl.BlockSpec((blk, c), lambda i: (i, 0)),
        out_shape=jax.ShapeDtypeStruct((n_pad, c), jnp.float32),
    )(a_p, w, b.reshape(1, c))
    return out[:n]


def _sage(x, src, dst, Wl, bl, Wr, n):
    msg = x[src]
    agg = jax.ops.segment_sum(msg, dst, num_segments=n)
    cnt = jax.ops.segment_sum(jnp.ones((src.shape[0],), x.dtype), dst,
                              num_segments=n)
    mean = agg / jnp.maximum(cnt, 1.0)[:, None]
    return mean @ Wl + bl + x @ Wr


def kernel(x, edge_index, missing_indices, Wl1, bl1, Wr1, Wl2, bl2, Wr2,
           Wp, bp, Wgl, bgl, Wgr, Wd1, bd1, Wd2, bd2):
    src = edge_index[0]
    dst = edge_index[1]
    n = x.shape[0]
    h = jax.nn.relu(_sage(x, src, dst, Wgl, bgl, Wgr, n))
    gen_feats = jax.nn.relu(h @ Wd1 + bd1) @ Wd2 + bd2
    new_feats = gen_feats[missing_indices]
    x_aug = jnp.concatenate([x, new_feats], axis=0)
    m = missing_indices.shape[0]
    new_node_indices = jnp.arange(n, n + m, dtype=missing_indices.dtype)
    row = jnp.concatenate([missing_indices, new_node_indices])
    col = jnp.concatenate([new_node_indices, missing_indices])
    src_aug = jnp.concatenate([src, row])
    dst_aug = jnp.concatenate([dst, col])
    n_aug = n + m
    h1 = jax.nn.relu(_sage(x_aug, src_aug, dst_aug, Wl1, bl1, Wr1, n_aug))
    h2 = jax.nn.relu(_sage(h1, src_aug, dst_aug, Wl2, bl2, Wr2, n_aug))
    return _proj(h2, Wp, bp)


# speedup vs baseline: 7.1505x; 7.1505x over previous
import functools

import jax
import jax.numpy as jnp
from jax import lax
from jax.experimental import pallas as pl
from jax.experimental.pallas import tpu as pltpu
from jax.experimental.pallas import tpu_sc as plsc

NP = 10240
RPS = NP // 16
TRASH = NP - 1
CH = 128
BLK = 256
F32 = jnp.float32


def _sc_mesh():
    return plsc.VectorSubcoreMesh(core_axis_name="c", subcore_axis_name="s")


@functools.lru_cache(maxsize=None)
def _segsum(nt, d, e):
    nch = e // CH
    dd = d // 16

    @functools.partial(
        pl.kernel,
        mesh=_sc_mesh(),
        out_type=[
            jax.ShapeDtypeStruct((2 * NP, d), F32),
            jax.ShapeDtypeStruct((2 * NP,), F32),
        ],
        scratch_types=[
            pltpu.VMEM((CH,), jnp.int32),
            pltpu.VMEM((CH,), jnp.int32),
            pltpu.VMEM((CH, d), F32),
            pltpu.VMEM((CH,), F32),
            pltpu.VMEM_SHARED((NP, d), F32),
            pltpu.VMEM_SHARED((NP,), F32),
            pltpu.SemaphoreType.DMA,
        ],
    )
    def k(table, srcl, dstl, out, cnt_out, src_v, dst_v, rows_v, ones_v,
          acc_sh, cnt_sh, sem):
        c = lax.axis_index("c")
        s = lax.axis_index("s")
        w = s * 2 + c

        def zero_body(i, carry):
            rows_v[i // dd, pl.ds((i % dd) * 16, 16)] = jnp.zeros((16,), F32)
            return carry

        lax.fori_loop(0, CH * dd, zero_body, 0)
        base = s * RPS
        for j in range(RPS // CH):
            pltpu.sync_copy(rows_v, acc_sh.at[pl.ds(base + j * CH, CH)])
            pltpu.sync_copy(rows_v.at[0], cnt_sh.at[pl.ds(base + j * CH, CH)])
        for j in range(CH // 16):
            ones_v[pl.ds(j * 16, 16)] = jnp.ones((16,), F32)
        plsc.subcore_barrier()

        n_w = nch // 32 + jnp.where(w < nch % 32, 1, 0).astype(jnp.int32)

        def body(i, carry):
            b = (w + i * 32) * CH
            pltpu.sync_copy(srcl.at[pl.ds(b, CH)], src_v)
            pltpu.sync_copy(dstl.at[pl.ds(b, CH)], dst_v)
            pltpu.async_copy(table.at[src_v], rows_v, sem).wait()
            pltpu.sync_copy(rows_v, acc_sh.at[dst_v], add=True)
            pltpu.sync_copy(ones_v, cnt_sh.at[dst_v], add=True)
            return carry

        lax.fori_loop(0, n_w, body, 0)
        plsc.subcore_barrier()
        ob = c * NP + base
        pltpu.sync_copy(acc_sh.at[pl.ds(base, RPS)], out.at[pl.ds(ob, RPS)])
        pltpu.sync_copy(cnt_sh.at[pl.ds(base, RPS)],
                        cnt_out.at[pl.ds(ob, RPS)])

    return k


@functools.lru_cache(maxsize=None)
def _gather(nt, d):
    bpw = 1024 // 32

    @functools.partial(
        pl.kernel,
        mesh=_sc_mesh(),
        out_type=jax.ShapeDtypeStruct((1024, d), F32),
        scratch_types=[
            pltpu.VMEM((bpw,), jnp.int32),
            pltpu.VMEM((bpw, d), F32),
            pltpu.SemaphoreType.DMA,
        ],
    )
    def k(table, idx, out, idx_v, rows_v, sem):
        w = lax.axis_index("s") * 2 + lax.axis_index("c")
        base = w * bpw
        pltpu.sync_copy(idx.at[pl.ds(base, bpw)], idx_v)
        pltpu.async_copy(table.at[idx_v], rows_v, sem).wait()
        pltpu.sync_copy(rows_v, out.at[pl.ds(base, bpw)])

    return k


def _mm(a, w):
    return jnp.dot(a, w, preferred_element_type=F32)


def _gen_body(aggA, aggB, cntA, cntB, xb, wgl, bgl, wgr, wd1, bd1, wd2, bd2,
              gen_o):
    cnt = cntA[...] + cntB[...]
    mean0 = (aggA[...] + aggB[...]) / jnp.maximum(cnt, 1.0)
    h = jnp.maximum(_mm(mean0, wgl[...]) + bgl[...] + _mm(xb[...], wgr[...]),
                    0.0)
    t = jnp.maximum(_mm(h, wd1[...]) + bd1[...], 0.0)
    gen_o[...] = _mm(t, wd2[...]) + bd2[...]


def _conv1_body(aggA, aggB, e1A, e1B, cntA, cntB, kA, kB, xb, wl1, bl1, wr1,
                h1lo_o, h1hi_o, den_o):
    den = jnp.maximum(cntA[...] + cntB[...] + kA[...] + kB[...], 1.0)
    den_r = 1.0 / den
    mean1 = (aggA[...] + aggB[...] + e1A[...] + e1B[...]) * den_r
    h1 = jnp.maximum(_mm(mean1, wl1[...]) + bl1[...] + _mm(xb[...], wr1[...]),
                     0.0)
    h1lo_o[...] = h1[:, :128]
    h1hi_o[...] = h1[:, 128:]
    den_o[...] = den_r


def _new1_body(xm, gm, wl1, bl1, wr1, lo_o, hi_o):
    h1n = jnp.maximum(_mm(xm[...], wl1[...]) + bl1[...] +
                      _mm(gm[...], wr1[...]), 0.0)
    lo_o[...] = h1n[:, :128]
    hi_o[...] = h1n[:, 128:]


def _conv2_body(aloA, aloB, ahiA, ahiB, eloA, eloB, ehiA, ehiB, den, h1lo,
                h1hi, wl2, bl2, wr2, wp, bp, out_o):
    d = den[...]
    mlo = (aloA[...] + aloB[...] + eloA[...] + eloB[...]) * d
    mhi = (ahiA[...] + ahiB[...] + ehiA[...] + ehiB[...]) * d
    wl2v = wl2[...]
    wr2v = wr2[...]
    h2 = jnp.maximum(
        _mm(mlo, wl2v[:128]) + _mm(mhi, wl2v[128:]) + bl2[...] +
        _mm(h1lo[...], wr2v[:128]) + _mm(h1hi[...], wr2v[128:]), 0.0)
    out_o[...] = _mm(h2, wp[...]) + bp[...]


def _new2_body(h1mlo, h1mhi, h1nlo, h1nhi, wl2, bl2, wr2, wp, bp, out_o):
    wl2v = wl2[...]
    wr2v = wr2[...]
    h2n = jnp.maximum(
        _mm(h1mlo[...], wl2v[:128]) + _mm(h1mhi[...], wl2v[128:]) + bl2[...] +
        _mm(h1nlo[...], wr2v[:128]) + _mm(h1nhi[...], wr2v[128:]), 0.0)
    out_o[...] = _mm(h2n, wp[...]) + bp[...]


def _row_spec(w, two_part):
    nb = NP // BLK
    if two_part == 0:
        return pl.BlockSpec((BLK, w), lambda i: (i, 0))
    return pl.BlockSpec((BLK, w), lambda i, nb=nb: (i + nb, 0))


def _full_spec(shape):
    nd = len(shape)
    return pl.BlockSpec(shape, lambda i: (0,) * nd)


def kernel(x, edge_index, missing_indices, Wl1, bl1, Wr1, Wl2, bl2, Wr2,
           Wp, bp, Wgl, bgl, Wgr, Wd1, bd1, Wd2, bd2):
    n, dx = x.shape
    e = edge_index.shape[1]
    m = missing_indices.shape[0]
    src = edge_index[0].astype(jnp.int32)
    dst = edge_index[1].astype(jnp.int32)
    midx = missing_indices.astype(jnp.int32)
    mp = 1024
    x_pad = jnp.pad(x, ((0, NP - n), (0, 0)))
    src_m = jnp.concatenate([midx, jnp.zeros((mp - m,), jnp.int32)])
    dst_m = jnp.concatenate([midx, jnp.full((mp - m,), TRASH, jnp.int32)])

    bgl_r = bgl.reshape(1, -1)
    bd1_r = bd1.reshape(1, -1)
    bd2_r = bd2.reshape(1, -1)
    bl1_r = bl1.reshape(1, -1)
    bl2_r = bl2.reshape(1, -1)
    bp_r = bp.reshape(1, -1)

    agg, cnt = _segsum(NP, 128, e)(x_pad, src, dst)
    cnt2 = cnt.reshape(2 * NP, 1)

    nb = NP // BLK
    gen = pl.pallas_call(
        _gen_body,
        grid=(nb,),
        in_specs=[
            _row_spec(128, 0), _row_spec(128, 1),
            _row_spec(1, 0), _row_spec(1, 1),
            _row_spec(128, 0),
            _full_spec((128, 256)), _full_spec((1, 256)),
            _full_spec((128, 256)),
            _full_spec((256, 256)), _full_spec((1, 256)),
            _full_spec((256, 128)), _full_spec((1, 128)),
        ],
        out_specs=_row_spec(128, 0),
        out_shape=jax.ShapeDtypeStruct((NP, 128), F32),
    )(agg, agg, cnt2, cnt2, x_pad, Wgl, bgl_r, Wgr, Wd1, bd1_r, Wd2, bd2_r)

    xm = _gather(NP, 128)(x_pad, src_m)
    gm = _gather(NP, 128)(gen, src_m)
    e1, kcnt = _segsum(NP, 128, mp)(gen, src_m, dst_m)
    k2 = kcnt.reshape(2 * NP, 1)

    h1lo, h1hi, den_r = pl.pallas_call(
        _conv1_body,
        grid=(nb,),
        in_specs=[
            _row_spec(128, 0), _row_spec(128, 1),
            _row_spec(128, 0), _row_spec(128, 1),
            _row_spec(1, 0), _row_spec(1, 1),
            _row_spec(1, 0), _row_spec(1, 1),
            _row_spec(128, 0),
            _full_spec((128, 256)), _full_spec((1, 256)),
            _full_spec((128, 256)),
        ],
        out_specs=[_row_spec(128, 0), _row_spec(128, 0), _row_spec(1, 0)],
        out_shape=[
            jax.ShapeDtypeStruct((NP, 128), F32),
            jax.ShapeDtypeStruct((NP, 128), F32),
            jax.ShapeDtypeStruct((NP, 1), F32),
        ],
    )(agg, agg, e1, e1, cnt2, cnt2, k2, k2, x_pad, Wl1, bl1_r, Wr1)

    h1nlo, h1nhi = pl.pallas_call(
        _new1_body,
        grid=(mp // BLK,),
        in_specs=[
            _row_spec(128, 0), _row_spec(128, 0),
            _full_spec((128, 256)), _full_spec((1, 256)),
            _full_spec((128, 256)),
        ],
        out_specs=[_row_spec(128, 0), _row_spec(128, 0)],
        out_shape=[
            jax.ShapeDtypeStruct((mp, 128), F32),
            jax.ShapeDtypeStruct((mp, 128), F32),
        ],
    )(xm, gm, Wl1, bl1_r, Wr1)

    a2lo, _ = _segsum(NP, 128, e)(h1lo, src, dst)
    a2hi, _ = _segsum(NP, 128, e)(h1hi, src, dst)
    ar = jnp.arange(mp, dtype=jnp.int32)
    e2lo, _ = _segsum(mp, 128, mp)(h1nlo, ar, dst_m)
    e2hi, _ = _segsum(mp, 128, mp)(h1nhi, ar, dst_m)
    h1mlo = _gather(NP, 128)(h1lo, src_m)
    h1mhi = _gather(NP, 128)(h1hi, src_m)

    out_main = pl.pallas_call(
        _conv2_body,
        grid=(nb,),
        in_specs=[
            _row_spec(128, 0), _row_spec(128, 1),
            _row_spec(128, 0), _row_spec(128, 1),
            _row_spec(128, 0), _row_spec(128, 1),
            _row_spec(128, 0), _row_spec(128, 1),
            _row_spec(1, 0),
            _row_spec(128, 0), _row_spec(128, 0),
            _full_spec((256, 256)), _full_spec((1, 256)),
            _full_spec((256, 256)),
            _full_spec((256, 64)), _full_spec((1, 64)),
        ],
        out_specs=_row_spec(64, 0),
        out_shape=jax.ShapeDtypeStruct((NP, 64), F32),
    )(a2lo, a2lo, a2hi, a2hi, e2lo, e2lo, e2hi, e2hi, den_r, h1lo, h1hi,
      Wl2, bl2_r, Wr2, Wp, bp_r)

    out_new = pl.pallas_call(
        _new2_body,
        grid=(mp // BLK,),
        in_specs=[
            _row_spec(128, 0), _row_spec(128, 0),
            _row_spec(128, 0), _row_spec(128, 0),
            _full_spec((256, 256)), _full_spec((1, 256)),
            _full_spec((256, 256)),
            _full_spec((256, 64)), _full_spec((1, 64)),
        ],
        out_specs=_row_spec(64, 0),
        out_shape=jax.ShapeDtypeStruct((mp, 64), F32),
    )(h1mlo, h1mhi, h1nlo, h1nhi, Wl2, bl2_r, Wr2, Wp, bp_r)

    return jnp.concatenate([out_main[:n], out_new[:m]], axis=0)
